# baseline (speedup 1.0000x reference)
import functools

import jax
import jax.numpy as jnp
from jax import lax
from jax.experimental import pallas as pl
from jax.experimental.pallas import tpu as pltpu
from jax.experimental.pallas import tpu_sc as plsc

MAX_PT = 1000000
EMBED_DIM = 32
BATCH = 16384

NC = 2
NS = 16
NW = NC * NS
B_PER_W = BATCH // NW
CHUNK = 128
N_CHUNK = B_PER_W // CHUNK
E_PER_W = B_PER_W * EMBED_DIM
GRP = 6

_mesh = plsc.VectorSubcoreMesh(core_axis_name="c", subcore_axis_name="s")


@functools.partial(
    pl.kernel,
    mesh=_mesh,
    out_type=jax.ShapeDtypeStruct((BATCH * EMBED_DIM,), jnp.float32),
    compiler_params=pltpu.CompilerParams(use_tc_tiling_on_sc=False,
                                         needs_layout_passes=False),
    scratch_types=[
        pltpu.VMEM((EMBED_DIM, B_PER_W), jnp.int32),
        pltpu.VMEM((EMBED_DIM, B_PER_W), jnp.float32),
        pltpu.VMEM((E_PER_W,), jnp.float32),
        pltpu.SemaphoreType.DMA,
    ],
)
def _gather_kernel(flat_hbm, fidx_hbm, out_hbm, fidx_v, cols_v, out_v, sem):
    wid = lax.axis_index("s") * NC + lax.axis_index("c")
    pltpu.sync_copy(fidx_hbm.at[:, pl.ds(wid * B_PER_W, B_PER_W)], fidx_v)
    copies = []
    for j in range(EMBED_DIM):
        for c in range(N_CHUNK):
            copies.append(
                pltpu.async_copy(
                    flat_hbm.at[fidx_v.at[j].at[pl.ds(c * CHUNK, CHUNK)]],
                    cols_v.at[j].at[pl.ds(c * CHUNK, CHUNK)],
                    sem,
                )
            )
    for cp in copies:
        cp.wait()
    lanes = lax.iota(jnp.int32, 16)

    def body(k, carry):
        i_vec = lanes + k * 16
        o_vec = i_vec * EMBED_DIM
        for j in range(EMBED_DIM):
            v = plsc.load_gather(cols_v,
                                 [jnp.full((16,), j, jnp.int32), i_vec])
            plsc.store_scatter(out_v, [o_vec + j], v)
        return carry

    lax.fori_loop(0, B_PER_W // 16, body, 0)
    pltpu.sync_copy(out_v, out_hbm.at[pl.ds(wid * E_PER_W, E_PER_W)])


def kernel(pt_id, theta_h_weight):
    flat = theta_h_weight.T.reshape(MAX_PT * EMBED_DIM)
    idx = jnp.clip(pt_id.astype(jnp.int32), 0, MAX_PT - 1)
    fidx = (jnp.arange(EMBED_DIM, dtype=jnp.int32)[:, None] * MAX_PT
            + idx[None, :])
    out_flat = _gather_kernel(flat, fidx)
    return out_flat.reshape(BATCH, EMBED_DIM)


# speedup vs baseline: 4.9411x; 4.9411x over previous
import functools

import jax
import jax.numpy as jnp
from jax import lax
from jax.experimental import pallas as pl
from jax.experimental.pallas import tpu as pltpu
from jax.experimental.pallas import tpu_sc as plsc

MAX_PT = 1000000
EMBED_DIM = 32
BATCH = 16384

NC = 2
NS = 16
NW = NC * NS
B_PER_W = BATCH // NW
CHUNK = 128
N_CHUNK = B_PER_W // CHUNK

_mesh = plsc.VectorSubcoreMesh(core_axis_name="c", subcore_axis_name="s")


@functools.partial(
    pl.kernel,
    mesh=_mesh,
    out_type=jax.ShapeDtypeStruct((BATCH, EMBED_DIM), jnp.float32),
    compiler_params=pltpu.CompilerParams(use_tc_tiling_on_sc=False),
    scratch_types=[
        pltpu.VMEM((N_CHUNK, CHUNK), jnp.int32),
        pltpu.VMEM((B_PER_W, EMBED_DIM), jnp.float32),
        pltpu.SemaphoreType.DMA,
    ],
)
def _gather_kernel(table_hbm, idx_hbm, out_hbm, idx_v, rows_v, sem):
    wid = lax.axis_index("s") * NC + lax.axis_index("c")
    base = wid * B_PER_W
    pltpu.sync_copy(idx_hbm.at[wid], idx_v)
    copies = []
    for j in range(N_CHUNK):
        copies.append(
            pltpu.async_copy(
                table_hbm.at[idx_v.at[j]],
                rows_v.at[pl.ds(j * CHUNK, CHUNK), :],
                sem,
            )
        )
    for c in copies:
        c.wait()
    pltpu.sync_copy(rows_v, out_hbm.at[pl.ds(base, B_PER_W)])


def kernel(pt_id, theta_h_weight):
    idx = jnp.clip(pt_id.astype(jnp.int32), 0, MAX_PT - 1)
    return _gather_kernel(theta_h_weight, idx.reshape(NW, N_CHUNK, CHUNK))
